# baseline (speedup 1.0000x reference)
import numpy as np

import jax
import jax.numpy as jnp
from jax.experimental import pallas as pl
from jax.experimental.pallas import tpu as pltpu
from jax.experimental.shard_map import shard_map
from jax.sharding import Mesh, PartitionSpec as P

Q = 1024
D = 512
K_TOTAL = 100000
N_DEV = 2
K_SHARD = K_TOTAL // N_DEV
K_TILE = 5000
N_TILES = K_SHARD // K_TILE
TOPK = 5
LANES = 128
SUBL = 8
UNROLL = 5

NEG_INF = float("-inf")
BIG_I32 = 2**31 - 1


def _topk_kernel(z_ref, y_ref, vals_out_ref, idx_out_ref,
                 st_ref, run_v_ref, run_i_ref, zn_ref):
    k = pl.program_id(0)

    @pl.when(k == 0)
    def _init():
        run_v_ref[...] = jnp.full((SUBL, Q), NEG_INF, dtype=jnp.float32)
        run_i_ref[...] = jnp.zeros((SUBL, Q), dtype=jnp.int32)
        z = z_ref[...]
        zn_ref[...] = z / jnp.sqrt(jnp.sum(z * z, axis=1, keepdims=True))

    st_ref[...] = jax.lax.dot_general(
        y_ref[...], zn_ref[...],
        dimension_numbers=(((1,), (1,)), ((), ())),
        preferred_element_type=jnp.float32,
    )

    base = k * K_TILE
    iota_s = jax.lax.broadcasted_iota(jnp.int32, (SUBL, LANES), 0)

    for g in range(Q // LANES):
        lo = g * LANES

        def body(r, carry, lo=lo):
            avs = list(carry[:TOPK])
            ais = list(carry[TOPK:])
            for u in range(UNROLL):
                row = (r * UNROLL + u) * SUBL
                v = st_ref[pl.ds(row, SUBL), lo:lo + LANES]
                iv = iota_s + (base + row)
                for j in range(TOPK):
                    gt = v > avs[j]
                    nav = jnp.where(gt, v, avs[j])
                    nai = jnp.where(gt, iv, ais[j])
                    v = jnp.where(gt, avs[j], v)
                    iv = jnp.where(gt, ais[j], iv)
                    avs[j] = nav
                    ais[j] = nai
            return tuple(avs) + tuple(ais)

        init = (tuple(jnp.full((SUBL, LANES), NEG_INF, dtype=jnp.float32)
                      for _ in range(TOPK))
                + tuple(jnp.zeros((SUBL, LANES), dtype=jnp.int32)
                        for _ in range(TOPK)))
        carry = jax.lax.fori_loop(0, K_TILE // SUBL // UNROLL, body, init,
                                  unroll=False)
        avs = list(carry[:TOPK])
        ais = list(carry[TOPK:])

        v_all = jnp.concatenate(avs + [run_v_ref[:, lo:lo + LANES]], axis=0)
        i_all = jnp.concatenate(ais + [run_i_ref[:, lo:lo + LANES]], axis=0)

        ms = []
        idxs = []
        for _ in range(TOPK):
            m = jnp.max(v_all, axis=0, keepdims=True)
            cand = jnp.where(v_all == m, i_all, BIG_I32)
            a = jnp.min(cand, axis=0, keepdims=True)
            v_all = jnp.where(cand == a, NEG_INF, v_all)
            ms.append(m)
            idxs.append(a)
        run_v_ref[:, lo:lo + LANES] = jnp.concatenate(
            ms + [ms[-1]] * (SUBL - TOPK), axis=0)
        run_i_ref[:, lo:lo + LANES] = jnp.concatenate(
            idxs + [idxs[-1]] * (SUBL - TOPK), axis=0)

    @pl.when(k == N_TILES - 1)
    def _finish():
        vals_out_ref[...] = run_v_ref[...]
        idx_out_ref[...] = run_i_ref[...]


def _merge_kernel(v_ref, i_ref, vals_out_ref, idx_out_ref):
    v_all = jnp.concatenate([v_ref[0], v_ref[1]], axis=0)
    i_all = jnp.concatenate([i_ref[0], i_ref[1]], axis=0)
    ms = []
    idxs = []
    for _ in range(TOPK):
        m = jnp.max(v_all, axis=0, keepdims=True)
        cand = jnp.where(v_all == m, i_all, BIG_I32)
        a = jnp.min(cand, axis=0, keepdims=True)
        v_all = jnp.where(cand == a, NEG_INF, v_all)
        ms.append(m)
        idxs.append(a)
    vals_out_ref[...] = jnp.concatenate(ms + [ms[-1]] * (SUBL - TOPK), axis=0)
    idx_out_ref[...] = jnp.concatenate(idxs + [idxs[-1]] * (SUBL - TOPK),
                                       axis=0)


def _shard_topk(z, y):
    vals_t, idx_t = pl.pallas_call(
        _topk_kernel,
        grid=(N_TILES,),
        in_specs=[
            pl.BlockSpec((Q, D), lambda k: (0, 0)),
            pl.BlockSpec((K_TILE, D), lambda k: (k, 0)),
        ],
        out_specs=[
            pl.BlockSpec((SUBL, Q), lambda k: (0, 0)),
            pl.BlockSpec((SUBL, Q), lambda k: (0, 0)),
        ],
        out_shape=[
            jax.ShapeDtypeStruct((SUBL, Q), jnp.float32),
            jax.ShapeDtypeStruct((SUBL, Q), jnp.int32),
        ],
        scratch_shapes=[
            pltpu.VMEM((K_TILE, Q), jnp.float32),
            pltpu.VMEM((SUBL, Q), jnp.float32),
            pltpu.VMEM((SUBL, Q), jnp.int32),
            pltpu.VMEM((Q, D), jnp.float32),
        ],
    )(z, y)
    idx_t = idx_t + jax.lax.axis_index("x").astype(jnp.int32) * K_SHARD
    g_v = jax.lax.all_gather(vals_t, "x")
    g_i = jax.lax.all_gather(idx_t, "x")
    vals_m, idx_m = pl.pallas_call(
        _merge_kernel,
        out_shape=[
            jax.ShapeDtypeStruct((SUBL, Q), jnp.float32),
            jax.ShapeDtypeStruct((SUBL, Q), jnp.int32),
        ],
    )(g_v, g_i)
    return vals_m, idx_m


@jax.jit
def kernel(Z, Y):
    mesh = Mesh(np.array(jax.devices()[:N_DEV]), ("x",))
    f = shard_map(
        _shard_topk, mesh=mesh,
        in_specs=(P(None, None), P("x", None)),
        out_specs=(P(None, None), P(None, None)),
        check_rep=False,
    )
    vals_t, idx_t = f(Z, Y)
    return vals_t[:TOPK].T, idx_t[:TOPK].T


# speedup vs baseline: 1.6312x; 1.6312x over previous
import jax
import jax.numpy as jnp
from jax.experimental import pallas as pl
from jax.experimental.pallas import tpu as pltpu

Q = 1024
D = 512
K_TOTAL = 100000
K_TILE = 4000
N_TILES = K_TOTAL // K_TILE
TOPK = 5
LANES = 128
SUBL = 8
UNROLL = 20

NEG_INF = float("-inf")
BIG_I32 = 2**31 - 1


def _topk_kernel(z_ref, y_ref, vals_out_ref, idx_out_ref,
                 st_ref, run_v_ref, run_i_ref, zn_ref):
    k = pl.program_id(0)

    @pl.when(k == 0)
    def _init():
        run_v_ref[...] = jnp.full((SUBL, Q), NEG_INF, dtype=jnp.float32)
        run_i_ref[...] = jnp.zeros((SUBL, Q), dtype=jnp.int32)
        z = z_ref[...]
        zn_ref[...] = z / jnp.sqrt(jnp.sum(z * z, axis=1, keepdims=True))

    st_ref[...] = jax.lax.dot_general(
        y_ref[...], zn_ref[...],
        dimension_numbers=(((1,), (1,)), ((), ())),
        preferred_element_type=jnp.float32,
    )

    base = k * K_TILE
    iota_s = jax.lax.broadcasted_iota(jnp.int32, (SUBL, LANES), 0)

    for g in range(Q // LANES):
        lo = g * LANES

        def body(r, carry, lo=lo):
            avs = list(carry[:TOPK])
            ais = list(carry[TOPK:])
            for u in range(UNROLL):
                row = (r * UNROLL + u) * SUBL
                v = st_ref[pl.ds(row, SUBL), lo:lo + LANES]
                iv = iota_s + (base + row)
                for j in range(TOPK):
                    gt = v > avs[j]
                    nav = jnp.where(gt, v, avs[j])
                    nai = jnp.where(gt, iv, ais[j])
                    v = jnp.where(gt, avs[j], v)
                    iv = jnp.where(gt, ais[j], iv)
                    avs[j] = nav
                    ais[j] = nai
            return tuple(avs) + tuple(ais)

        init = (tuple(jnp.full((SUBL, LANES), NEG_INF, dtype=jnp.float32)
                      for _ in range(TOPK))
                + tuple(jnp.zeros((SUBL, LANES), dtype=jnp.int32)
                        for _ in range(TOPK)))
        carry = jax.lax.fori_loop(0, K_TILE // SUBL // UNROLL, body, init,
                                  unroll=False)
        avs = list(carry[:TOPK])
        ais = list(carry[TOPK:])

        v_all = jnp.concatenate(avs + [run_v_ref[:, lo:lo + LANES]], axis=0)
        i_all = jnp.concatenate(ais + [run_i_ref[:, lo:lo + LANES]], axis=0)

        ms = []
        idxs = []
        for _ in range(TOPK):
            m = jnp.max(v_all, axis=0, keepdims=True)
            cand = jnp.where(v_all == m, i_all, BIG_I32)
            a = jnp.min(cand, axis=0, keepdims=True)
            v_all = jnp.where(cand == a, NEG_INF, v_all)
            ms.append(m)
            idxs.append(a)
        run_v_ref[:, lo:lo + LANES] = jnp.concatenate(
            ms + [ms[-1]] * (SUBL - TOPK), axis=0)
        run_i_ref[:, lo:lo + LANES] = jnp.concatenate(
            idxs + [idxs[-1]] * (SUBL - TOPK), axis=0)

    @pl.when(k == N_TILES - 1)
    def _finish():
        vals_out_ref[...] = run_v_ref[...]
        idx_out_ref[...] = run_i_ref[...]


@jax.jit
def kernel(Z, Y):
    vals_t, idx_t = pl.pallas_call(
        _topk_kernel,
        grid=(N_TILES,),
        in_specs=[
            pl.BlockSpec((Q, D), lambda k: (0, 0)),
            pl.BlockSpec((K_TILE, D), lambda k: (k, 0)),
        ],
        out_specs=[
            pl.BlockSpec((SUBL, Q), lambda k: (0, 0)),
            pl.BlockSpec((SUBL, Q), lambda k: (0, 0)),
        ],
        out_shape=[
            jax.ShapeDtypeStruct((SUBL, Q), jnp.float32),
            jax.ShapeDtypeStruct((SUBL, Q), jnp.int32),
        ],
        scratch_shapes=[
            pltpu.VMEM((K_TILE, Q), jnp.float32),
            pltpu.VMEM((SUBL, Q), jnp.float32),
            pltpu.VMEM((SUBL, Q), jnp.int32),
            pltpu.VMEM((Q, D), jnp.float32),
        ],
    )(Z, Y)
    return vals_t[:TOPK].T, idx_t[:TOPK].T


# speedup vs baseline: 1.6869x; 1.0341x over previous
import jax
import jax.numpy as jnp
from jax.experimental import pallas as pl
from jax.experimental.pallas import tpu as pltpu

Q = 1024
D = 512
K_TOTAL = 100000
K_TILE = 4000
N_TILES = K_TOTAL // K_TILE
TOPK = 5
LANES = 128
SUBL = 8
UNROLL = 50

NEG_INF = float("-inf")
BIG_I32 = 2**31 - 1


def _topk_kernel(z_ref, y_ref, vals_out_ref, idx_out_ref,
                 st_ref, run_v_ref, run_i_ref, zn_ref):
    k = pl.program_id(0)

    @pl.when(k == 0)
    def _init():
        run_v_ref[...] = jnp.full((SUBL, Q), NEG_INF, dtype=jnp.float32)
        run_i_ref[...] = jnp.zeros((SUBL, Q), dtype=jnp.int32)
        z = z_ref[...]
        zn_ref[...] = z / jnp.sqrt(jnp.sum(z * z, axis=1, keepdims=True))

    st_ref[...] = jax.lax.dot_general(
        y_ref[...], zn_ref[...],
        dimension_numbers=(((1,), (1,)), ((), ())),
        preferred_element_type=jnp.float32,
    )

    base = k * K_TILE
    iota_s = jax.lax.broadcasted_iota(jnp.int32, (SUBL, LANES), 0)

    for g in range(Q // LANES):
        lo = g * LANES

        def body(r, carry, lo=lo):
            avs = list(carry[:TOPK])
            ais = list(carry[TOPK:])
            for u in range(UNROLL):
                row = (r * UNROLL + u) * SUBL
                v = st_ref[pl.ds(row, SUBL), lo:lo + LANES]
                iv = iota_s + (base + row)
                for j in range(TOPK):
                    gt = v > avs[j]
                    nav = jnp.where(gt, v, avs[j])
                    nai = jnp.where(gt, iv, ais[j])
                    v = jnp.where(gt, avs[j], v)
                    iv = jnp.where(gt, ais[j], iv)
                    avs[j] = nav
                    ais[j] = nai
            return tuple(avs) + tuple(ais)

        init = (tuple(jnp.full((SUBL, LANES), NEG_INF, dtype=jnp.float32)
                      for _ in range(TOPK))
                + tuple(jnp.zeros((SUBL, LANES), dtype=jnp.int32)
                        for _ in range(TOPK)))
        carry = jax.lax.fori_loop(0, K_TILE // SUBL // UNROLL, body, init,
                                  unroll=False)
        avs = list(carry[:TOPK])
        ais = list(carry[TOPK:])

        v_all = jnp.concatenate(avs + [run_v_ref[:, lo:lo + LANES]], axis=0)
        i_all = jnp.concatenate(ais + [run_i_ref[:, lo:lo + LANES]], axis=0)

        ms = []
        idxs = []
        for _ in range(TOPK):
            m = jnp.max(v_all, axis=0, keepdims=True)
            cand = jnp.where(v_all == m, i_all, BIG_I32)
            a = jnp.min(cand, axis=0, keepdims=True)
            v_all = jnp.where(cand == a, NEG_INF, v_all)
            ms.append(m)
            idxs.append(a)
        run_v_ref[:, lo:lo + LANES] = jnp.concatenate(
            ms + [ms[-1]] * (SUBL - TOPK), axis=0)
        run_i_ref[:, lo:lo + LANES] = jnp.concatenate(
            idxs + [idxs[-1]] * (SUBL - TOPK), axis=0)

    @pl.when(k == N_TILES - 1)
    def _finish():
        vals_out_ref[...] = run_v_ref[...]
        idx_out_ref[...] = run_i_ref[...]


@jax.jit
def kernel(Z, Y):
    vals_t, idx_t = pl.pallas_call(
        _topk_kernel,
        grid=(N_TILES,),
        in_specs=[
            pl.BlockSpec((Q, D), lambda k: (0, 0)),
            pl.BlockSpec((K_TILE, D), lambda k: (k, 0)),
        ],
        out_specs=[
            pl.BlockSpec((SUBL, Q), lambda k: (0, 0)),
            pl.BlockSpec((SUBL, Q), lambda k: (0, 0)),
        ],
        out_shape=[
            jax.ShapeDtypeStruct((SUBL, Q), jnp.float32),
            jax.ShapeDtypeStruct((SUBL, Q), jnp.int32),
        ],
        scratch_shapes=[
            pltpu.VMEM((K_TILE, Q), jnp.float32),
            pltpu.VMEM((SUBL, Q), jnp.float32),
            pltpu.VMEM((SUBL, Q), jnp.int32),
            pltpu.VMEM((Q, D), jnp.float32),
        ],
    )(Z, Y)
    return vals_t[:TOPK].T, idx_t[:TOPK].T


# speedup vs baseline: 1.7116x; 1.0146x over previous
import jax
import jax.numpy as jnp
from jax.experimental import pallas as pl
from jax.experimental.pallas import tpu as pltpu

Q = 1024
D = 512
K_TOTAL = 100000
K_TILE = 4000
N_TILES = K_TOTAL // K_TILE
TOPK = 5
LANES = 128
SUBL = 8
UNROLL = 100

NEG_INF = float("-inf")
BIG_I32 = 2**31 - 1


def _topk_kernel(z_ref, y_ref, vals_out_ref, idx_out_ref,
                 st_ref, run_v_ref, run_i_ref, zn_ref):
    k = pl.program_id(0)

    @pl.when(k == 0)
    def _init():
        run_v_ref[...] = jnp.full((SUBL, Q), NEG_INF, dtype=jnp.float32)
        run_i_ref[...] = jnp.zeros((SUBL, Q), dtype=jnp.int32)
        z = z_ref[...]
        zn_ref[...] = z / jnp.sqrt(jnp.sum(z * z, axis=1, keepdims=True))

    st_ref[...] = jax.lax.dot_general(
        y_ref[...], zn_ref[...],
        dimension_numbers=(((1,), (1,)), ((), ())),
        preferred_element_type=jnp.float32,
    )

    base = k * K_TILE
    iota_s = jax.lax.broadcasted_iota(jnp.int32, (SUBL, LANES), 0)

    for g in range(Q // LANES):
        lo = g * LANES

        def body(r, carry, lo=lo):
            avs = list(carry[:TOPK])
            ais = list(carry[TOPK:])
            for u in range(UNROLL):
                row = (r * UNROLL + u) * SUBL
                v = st_ref[pl.ds(row, SUBL), lo:lo + LANES]
                iv = iota_s + (base + row)
                for j in range(TOPK):
                    gt = v > avs[j]
                    nav = jnp.where(gt, v, avs[j])
                    nai = jnp.where(gt, iv, ais[j])
                    v = jnp.where(gt, avs[j], v)
                    iv = jnp.where(gt, ais[j], iv)
                    avs[j] = nav
                    ais[j] = nai
            return tuple(avs) + tuple(ais)

        init = (tuple(jnp.full((SUBL, LANES), NEG_INF, dtype=jnp.float32)
                      for _ in range(TOPK))
                + tuple(jnp.zeros((SUBL, LANES), dtype=jnp.int32)
                        for _ in range(TOPK)))
        carry = jax.lax.fori_loop(0, K_TILE // SUBL // UNROLL, body, init,
                                  unroll=False)
        avs = list(carry[:TOPK])
        ais = list(carry[TOPK:])

        v_all = jnp.concatenate(avs + [run_v_ref[:, lo:lo + LANES]], axis=0)
        i_all = jnp.concatenate(ais + [run_i_ref[:, lo:lo + LANES]], axis=0)

        ms = []
        idxs = []
        for _ in range(TOPK):
            m = jnp.max(v_all, axis=0, keepdims=True)
            cand = jnp.where(v_all == m, i_all, BIG_I32)
            a = jnp.min(cand, axis=0, keepdims=True)
            v_all = jnp.where(cand == a, NEG_INF, v_all)
            ms.append(m)
            idxs.append(a)
        run_v_ref[:, lo:lo + LANES] = jnp.concatenate(
            ms + [ms[-1]] * (SUBL - TOPK), axis=0)
        run_i_ref[:, lo:lo + LANES] = jnp.concatenate(
            idxs + [idxs[-1]] * (SUBL - TOPK), axis=0)

    @pl.when(k == N_TILES - 1)
    def _finish():
        vals_out_ref[...] = run_v_ref[...]
        idx_out_ref[...] = run_i_ref[...]


@jax.jit
def kernel(Z, Y):
    vals_t, idx_t = pl.pallas_call(
        _topk_kernel,
        grid=(N_TILES,),
        in_specs=[
            pl.BlockSpec((Q, D), lambda k: (0, 0)),
            pl.BlockSpec((K_TILE, D), lambda k: (k, 0)),
        ],
        out_specs=[
            pl.BlockSpec((SUBL, Q), lambda k: (0, 0)),
            pl.BlockSpec((SUBL, Q), lambda k: (0, 0)),
        ],
        out_shape=[
            jax.ShapeDtypeStruct((SUBL, Q), jnp.float32),
            jax.ShapeDtypeStruct((SUBL, Q), jnp.int32),
        ],
        scratch_shapes=[
            pltpu.VMEM((K_TILE, Q), jnp.float32),
            pltpu.VMEM((SUBL, Q), jnp.float32),
            pltpu.VMEM((SUBL, Q), jnp.int32),
            pltpu.VMEM((Q, D), jnp.float32),
        ],
    )(Z, Y)
    return vals_t[:TOPK].T, idx_t[:TOPK].T
